# baseline (speedup 1.0000x reference)
import functools

import jax
import jax.numpy as jnp
from jax import lax
from jax.experimental import pallas as pl
from jax.experimental.pallas import tpu as pltpu
from jax.experimental.pallas import tpu_sc as plsc

_LANES = 128
_NW = 32


def _make_sc_segment_sum(n_nodes, d, rows_total, rows_per_worker, acc_rows,
                         rows_per_core0):
    mesh = plsc.VectorSubcoreMesh(core_axis_name="c", subcore_axis_name="s")
    zero_copies = acc_rows // (16 * _LANES)
    out_rows_per_tile = acc_rows // 16

    sup = 16
    assert rows_per_worker % sup == 0
    assert rows_per_core0 % sup == 0
    assert 2 * sup <= rows_per_core0 <= 2 * rows_per_worker - 2 * sup

    @functools.partial(
        pl.kernel,
        out_type=jax.ShapeDtypeStruct((2, acc_rows, d), jnp.float32),
        mesh=mesh,
        scratch_types=[pltpu.VMEM((sup, _LANES), jnp.int32) for _ in range(4)]
        + [pltpu.VMEM((_LANES, d), jnp.float32) for _ in range(2)]
        + [pltpu.VMEM_SHARED((acc_rows, d), jnp.float32)]
        + [pltpu.SemaphoreType.DMA for _ in range(8)],
    )
    def seg_sum(x_hbm, src_hbm, dst_hbm, out_hbm, *rest):
        sbuf, dbuf = rest[0:2], rest[2:4]
        rbuf = rest[4:6]
        acc = rest[6]
        isems, isemd = rest[7:9], rest[9:11]
        gsem, ssem = rest[11:13], rest[13:15]
        c = lax.axis_index("c")
        s = lax.axis_index("s")

        def zbody(i, carry):
            rbuf[0][i // 8, pl.ds((i % 8) * 16, 16)] = jnp.zeros((16,), jnp.float32)
            return carry

        with jax.named_scope("zero"):
            lax.fori_loop(0, _LANES * (d // 16), zbody, 0)
            for k in range(zero_copies):
                pltpu.sync_copy(rbuf[0], acc.at[pl.ds((s * zero_copies + k) * _LANES, _LANES)])
            plsc.subcore_barrier()

        def pipeline(base, nrows):
            nsup_l = nrows // sup

            def idx_start(q):
                sl = pl.ds(base + q * sup, sup)
                pltpu.async_copy(src_hbm.at[sl], sbuf[q & 1], isems[q & 1])
                pltpu.async_copy(dst_hbm.at[sl], dbuf[q & 1], isemd[q & 1])

            def idx_wait(q):
                sl = pl.ds(base + q * sup, sup)
                pltpu.make_async_copy(src_hbm.at[sl], sbuf[q & 1], isems[q & 1]).wait()
                pltpu.make_async_copy(dst_hbm.at[sl], dbuf[q & 1], isemd[q & 1]).wait()

            def start_gather(j):
                p, q, r = j & 1, (j // sup) & 1, j % sup
                pltpu.async_copy(x_hbm.at[sbuf[q].at[r]], rbuf[p], gsem[p])

            def wait_gather(j):
                p, q, r = j & 1, (j // sup) & 1, j % sup
                pltpu.make_async_copy(x_hbm.at[sbuf[q].at[r]], rbuf[p], gsem[p]).wait()

            def start_scatter(j):
                p, q, r = j & 1, (j // sup) & 1, j % sup
                pltpu.async_copy(rbuf[p], acc.at[dbuf[q].at[r]], ssem[p], add=True)

            def wait_scatter(j):
                p, q, r = j & 1, (j // sup) & 1, j % sup
                pltpu.make_async_copy(rbuf[p], acc.at[dbuf[q].at[r]], ssem[p]).wait()

            idx_start(0)
            if nsup_l > 1:
                idx_start(1)
            idx_wait(0)
            start_gather(0)
            start_gather(1)

            for j in range(nrows):
                wait_gather(j)
                start_scatter(j)
                wait_scatter(j)
                nj = j + 2
                if nj < nrows:
                    if nj % sup == 0:
                        idx_wait(nj // sup)
                    start_gather(nj)
                if j % sup == sup - 1 and j // sup + 2 < nsup_l:
                    idx_start(j // sup + 2)

        r0 = rows_per_core0
        r1 = rows_per_worker * 2 - r0

        with jax.named_scope("pipe"):
            @pl.when(c == 0)
            def _():
                pipeline(s * r0, r0)

            @pl.when(c == 1)
            def _():
                pipeline(16 * r0 + s * r1, r1)

            plsc.subcore_barrier()

        with jax.named_scope("dump"):
            rbase = s * out_rows_per_tile
            pltpu.sync_copy(
                acc.at[pl.ds(rbase, out_rows_per_tile)],
                out_hbm.at[c, pl.ds(rbase, out_rows_per_tile)],
            )

    return seg_sum


def _tc_combine(aggp, x, w_rel, b2d, w_root, relu):
    n, d = x.shape
    blk = 1000

    def body(a_ref, x_ref, wr_ref, wt_ref, b_ref, o_ref):
        a = a_ref[0] + a_ref[1]
        acc = lax.dot_general(a, wr_ref[...], (((1,), (1,)), ((), ())),
                              preferred_element_type=jnp.float32)
        acc = acc + lax.dot_general(x_ref[...], wt_ref[...], (((1,), (1,)), ((), ())),
                                    preferred_element_type=jnp.float32)
        acc = acc + b_ref[...]
        if relu:
            acc = jnp.maximum(acc, 0.0)
        o_ref[...] = acc

    return pl.pallas_call(
        body,
        grid=(n // blk,),
        in_specs=[
            pl.BlockSpec((2, blk, d), lambda i: (0, i, 0)),
            pl.BlockSpec((blk, d), lambda i: (i, 0)),
            pl.BlockSpec((d, d), lambda i: (0, 0)),
            pl.BlockSpec((d, d), lambda i: (0, 0)),
            pl.BlockSpec((1, d), lambda i: (0, 0)),
        ],
        out_specs=pl.BlockSpec((blk, d), lambda i: (i, 0)),
        out_shape=jax.ShapeDtypeStruct((n, d), jnp.float32),
    )(aggp, x, w_rel, w_root, b2d)


def kernel(x, edge_index, W1_rel, b1, W1_root, W2_rel, b2, W2_root):
    n, d = x.shape
    e = edge_index.shape[1]

    rows_per_worker = -(-e // (_LANES * _NW * 8)) * 8
    rows_total = rows_per_worker * _NW
    pad = rows_total * _LANES - e
    acc_rows = -(-(n + 1) // (16 * _LANES)) * (16 * _LANES)

    src = edge_index[0].astype(jnp.int32)
    dst = edge_index[1].astype(jnp.int32)
    if pad:
        src = jnp.concatenate([src, jnp.zeros((pad,), jnp.int32)])
        dst = jnp.concatenate(
            [dst, n + (jnp.arange(pad, dtype=jnp.int32) % (acc_rows - n))])
    src2d = src.reshape(rows_total, _LANES)
    dst2d = dst.reshape(rows_total, _LANES)

    rows_per_core0 = rows_per_worker
    seg_sum = _make_sc_segment_sum(n, d, rows_total, rows_per_worker, acc_rows,
                                   rows_per_core0)
    b1_2d = b1.reshape(1, d)
    b2_2d = b2.reshape(1, d)

    aggp1 = seg_sum(x, src2d, dst2d)
    h = _tc_combine(aggp1, x, W1_rel, b1_2d, W1_root, relu=True)
    aggp2 = seg_sum(h, src2d, dst2d)
    return _tc_combine(aggp2, h, W2_rel, b2_2d, W2_root, relu=False)


# speedup vs baseline: 3.6711x; 3.6711x over previous
import functools

import jax
import jax.numpy as jnp
from jax import lax
from jax.experimental import pallas as pl
from jax.experimental.pallas import tpu as pltpu
from jax.experimental.pallas import tpu_sc as plsc

_LANES = 128
_NW = 32


def _make_sc_segment_sum(n_nodes, d, rows_total, rows_per_worker, acc_rows,
                         rows_per_core0):
    mesh = plsc.VectorSubcoreMesh(core_axis_name="c", subcore_axis_name="s")
    zero_copies = acc_rows // (16 * _LANES)
    out_rows_per_tile = acc_rows // 16

    sup = 16
    assert rows_per_worker % sup == 0
    assert rows_per_core0 % sup == 0
    assert 2 * sup <= rows_per_core0 <= 2 * rows_per_worker - 2 * sup

    @functools.partial(
        pl.kernel,
        out_type=jax.ShapeDtypeStruct((2, acc_rows, d), jnp.float32),
        mesh=mesh,
        scratch_types=[pltpu.VMEM((sup, _LANES), jnp.int32) for _ in range(4)]
        + [pltpu.VMEM((_LANES, d), jnp.float32) for _ in range(2)]
        + [pltpu.VMEM_SHARED((acc_rows, d), jnp.float32)]
        + [pltpu.SemaphoreType.DMA for _ in range(8)],
    )
    def seg_sum(x_hbm, src_hbm, dst_hbm, out_hbm, *rest):
        sbuf, dbuf = rest[0:2], rest[2:4]
        rbuf = rest[4:6]
        acc = rest[6]
        isems, isemd = rest[7:9], rest[9:11]
        gsem, ssem = rest[11:13], rest[13:15]
        c = lax.axis_index("c")
        s = lax.axis_index("s")

        def zbody(i, carry):
            rbuf[0][i // 8, pl.ds((i % 8) * 16, 16)] = jnp.zeros((16,), jnp.float32)
            return carry

        with jax.named_scope("zero"):
            lax.fori_loop(0, _LANES * (d // 16), zbody, 0)
            for k in range(zero_copies):
                pltpu.sync_copy(rbuf[0], acc.at[pl.ds((s * zero_copies + k) * _LANES, _LANES)])
            plsc.subcore_barrier()

        def pipeline(base, nrows):
            nsup_l = nrows // sup

            def idx_start(q):
                sl = pl.ds(base + q * sup, sup)
                pltpu.async_copy(src_hbm.at[sl], sbuf[q & 1], isems[q & 1])
                pltpu.async_copy(dst_hbm.at[sl], dbuf[q & 1], isemd[q & 1])

            def idx_wait(q):
                sl = pl.ds(base + q * sup, sup)
                pltpu.make_async_copy(src_hbm.at[sl], sbuf[q & 1], isems[q & 1]).wait()
                pltpu.make_async_copy(dst_hbm.at[sl], dbuf[q & 1], isemd[q & 1]).wait()

            def start_gather(j):
                p, q, r = j & 1, (j // sup) & 1, j % sup
                pltpu.async_copy(x_hbm.at[sbuf[q].at[r]], rbuf[p], gsem[p])

            def wait_gather(j):
                p, q, r = j & 1, (j // sup) & 1, j % sup
                pltpu.make_async_copy(x_hbm.at[sbuf[q].at[r]], rbuf[p], gsem[p]).wait()

            def start_scatter(j):
                p, q, r = j & 1, (j // sup) & 1, j % sup
                pltpu.async_copy(rbuf[p], acc.at[dbuf[q].at[r]], ssem[p], add=True)

            def wait_scatter(j):
                p, q, r = j & 1, (j // sup) & 1, j % sup
                pltpu.make_async_copy(rbuf[p], acc.at[dbuf[q].at[r]], ssem[p]).wait()

            idx_start(0)
            if nsup_l > 1:
                idx_start(1)
            idx_wait(0)
            start_gather(0)
            start_gather(1)

            for j in range(nrows):
                wait_gather(j)
                start_scatter(j)
                wait_scatter(j)
                nj = j + 2
                if nj < nrows:
                    if nj % sup == 0:
                        idx_wait(nj // sup)
                    start_gather(nj)
                if j % sup == sup - 1 and j // sup + 2 < nsup_l:
                    idx_start(j // sup + 2)

        r0 = rows_per_core0
        r1 = rows_per_worker * 2 - r0

        with jax.named_scope("pipe"):
            @pl.when(c == 0)
            def _():
                pipeline(s * r0, r0)

            @pl.when(c == 1)
            def _():
                pipeline(16 * r0 + s * r1, r1)

        with jax.named_scope("bar"):
            plsc.subcore_barrier()

        with jax.named_scope("dump"):
            rbase = s * out_rows_per_tile
            pltpu.sync_copy(
                acc.at[pl.ds(rbase, out_rows_per_tile)],
                out_hbm.at[c, pl.ds(rbase, out_rows_per_tile)],
            )

    return seg_sum


def _tc_combine(aggp, x, w_rel, b2d, w_root, relu):
    n, d = x.shape
    blk = 1000

    def body(a_ref, x_ref, wr_ref, wt_ref, b_ref, o_ref):
        a = a_ref[0] + a_ref[1]
        acc = lax.dot_general(a, wr_ref[...], (((1,), (1,)), ((), ())),
                              preferred_element_type=jnp.float32)
        acc = acc + lax.dot_general(x_ref[...], wt_ref[...], (((1,), (1,)), ((), ())),
                                    preferred_element_type=jnp.float32)
        acc = acc + b_ref[...]
        if relu:
            acc = jnp.maximum(acc, 0.0)
        o_ref[...] = acc

    return pl.pallas_call(
        body,
        grid=(n // blk,),
        in_specs=[
            pl.BlockSpec((2, blk, d), lambda i: (0, i, 0)),
            pl.BlockSpec((blk, d), lambda i: (i, 0)),
            pl.BlockSpec((d, d), lambda i: (0, 0)),
            pl.BlockSpec((d, d), lambda i: (0, 0)),
            pl.BlockSpec((1, d), lambda i: (0, 0)),
        ],
        out_specs=pl.BlockSpec((blk, d), lambda i: (i, 0)),
        out_shape=jax.ShapeDtypeStruct((n, d), jnp.float32),
    )(aggp, x, w_rel, w_root, b2d)


def kernel(x, edge_index, W1_rel, b1, W1_root, W2_rel, b2, W2_root):
    n, d = x.shape
    e = edge_index.shape[1]

    rows_per_worker = -(-e // (_LANES * _NW * 8)) * 8
    rows_total = rows_per_worker * _NW
    pad = rows_total * _LANES - e
    acc_rows = -(-(n + 1) // (16 * _LANES)) * (16 * _LANES)

    src = edge_index[0].astype(jnp.int32)
    dst = edge_index[1].astype(jnp.int32)
    if pad:
        r = jnp.arange(pad, dtype=jnp.int32)
        src = jnp.concatenate([src, r % n])
        dst = jnp.concatenate([dst, n + (r % (acc_rows - n))])
    src2d = src.reshape(rows_total, _LANES)
    dst2d = dst.reshape(rows_total, _LANES)

    rows_per_core0 = rows_per_worker
    seg_sum = _make_sc_segment_sum(n, d, rows_total, rows_per_worker, acc_rows,
                                   rows_per_core0)
    b1_2d = b1.reshape(1, d)
    b2_2d = b2.reshape(1, d)

    aggp1 = seg_sum(x, src2d, dst2d)
    h = _tc_combine(aggp1, x, W1_rel, b1_2d, W1_root, relu=True)
    aggp2 = seg_sum(h, src2d, dst2d)
    return _tc_combine(aggp2, h, W2_rel, b2_2d, W2_root, relu=False)


# speedup vs baseline: 3.7605x; 1.0244x over previous
import functools

import jax
import jax.numpy as jnp
from jax import lax
from jax.experimental import pallas as pl
from jax.experimental.pallas import tpu as pltpu
from jax.experimental.pallas import tpu_sc as plsc

_LANES = 128
_NW = 32
_NRING = 4


def _make_sc_segment_sum(n_nodes, d, e_rows, acc_rows):
    mesh = plsc.VectorSubcoreMesh(core_axis_name="c", subcore_axis_name="s")
    zero_copies = acc_rows // (16 * _LANES)
    out_rows_per_tile = acc_rows // 16
    q, rmd = divmod(e_rows, _NW)
    assert q >= _NRING

    @functools.partial(
        pl.kernel,
        out_type=jax.ShapeDtypeStruct((2, acc_rows, d), jnp.float32),
        mesh=mesh,
        scratch_types=[pltpu.VMEM((_NRING, _LANES), jnp.int32) for _ in range(2)]
        + [pltpu.VMEM((_LANES, d), jnp.float32) for _ in range(2)]
        + [pltpu.VMEM_SHARED((acc_rows, d), jnp.float32)]
        + [pltpu.SemaphoreType.DMA for _ in range(2 * _NRING + 4)],
    )
    def seg_sum(x_hbm, src_hbm, dst_hbm, out_hbm, *rest):
        sring, dring = rest[0], rest[1]
        rbuf = rest[2:4]
        acc = rest[4]
        isems = rest[5:5 + _NRING]
        isemd = rest[5 + _NRING:5 + 2 * _NRING]
        gsem = rest[5 + 2 * _NRING:7 + 2 * _NRING]
        ssem = rest[7 + 2 * _NRING:]
        c = lax.axis_index("c")
        s = lax.axis_index("s")
        w = c * 16 + s

        with jax.named_scope("zero"):
            def zrow(i, carry):
                for k in range(d // 16):
                    rbuf[0][i, pl.ds(k * 16, 16)] = jnp.zeros((16,), jnp.float32)
                return carry

            lax.fori_loop(0, _LANES, zrow, 0)
            for k in range(zero_copies):
                pltpu.sync_copy(
                    rbuf[0], acc.at[pl.ds((s * zero_copies + k) * _LANES, _LANES)])
            plsc.subcore_barrier()

        def pipeline(base, nrows):
            def idx_start(j):
                t = j % _NRING
                sl = pl.ds((base + j) * _LANES, _LANES)
                pltpu.async_copy(src_hbm.at[sl], sring.at[t], isems[t])
                pltpu.async_copy(dst_hbm.at[sl], dring.at[t], isemd[t])

            def idx_wait(j):
                t = j % _NRING
                sl = pl.ds((base + j) * _LANES, _LANES)
                pltpu.make_async_copy(src_hbm.at[sl], sring.at[t], isems[t]).wait()
                pltpu.make_async_copy(dst_hbm.at[sl], dring.at[t], isemd[t]).wait()

            def start_gather(j):
                p, t = j & 1, j % _NRING
                pltpu.async_copy(x_hbm.at[sring.at[t]], rbuf[p], gsem[p])

            def wait_gather(j):
                p, t = j & 1, j % _NRING
                pltpu.make_async_copy(x_hbm.at[sring.at[t]], rbuf[p], gsem[p]).wait()

            def start_scatter(j):
                p, t = j & 1, j % _NRING
                pltpu.async_copy(rbuf[p], acc.at[dring.at[t]], ssem[p], add=True)

            def wait_scatter(j):
                p, t = j & 1, j % _NRING
                pltpu.make_async_copy(rbuf[p], acc.at[dring.at[t]], ssem[p]).wait()

            for t in range(_NRING):
                idx_start(t)
            idx_wait(0)
            start_gather(0)
            idx_wait(1)
            start_gather(1)

            for j in range(nrows):
                wait_gather(j)
                start_scatter(j)
                wait_scatter(j)
                if j + _NRING < nrows:
                    idx_start(j + _NRING)
                if j + 2 < nrows:
                    idx_wait(j + 2)
                    start_gather(j + 2)

        with jax.named_scope("pipe"):
            if rmd:
                @pl.when(w < rmd)
                def _():
                    pipeline(w * (q + 1), q + 1)

                @pl.when(w >= rmd)
                def _():
                    pipeline(w * q + rmd, q)
            else:
                pipeline(w * q, q)

        with jax.named_scope("bar"):
            plsc.subcore_barrier()

        with jax.named_scope("dump"):
            rbase = s * out_rows_per_tile
            pltpu.sync_copy(
                acc.at[pl.ds(rbase, out_rows_per_tile)],
                out_hbm.at[c, pl.ds(rbase, out_rows_per_tile)],
            )

    return seg_sum


def _tc_combine(aggp, x, w_rel, b2d, w_root, relu):
    n, d = x.shape
    blk = 1000

    def body(a_ref, x_ref, wr_ref, wt_ref, b_ref, o_ref):
        a = a_ref[0] + a_ref[1]
        acc = lax.dot_general(a, wr_ref[...], (((1,), (1,)), ((), ())),
                              preferred_element_type=jnp.float32)
        acc = acc + lax.dot_general(x_ref[...], wt_ref[...], (((1,), (1,)), ((), ())),
                                    preferred_element_type=jnp.float32)
        acc = acc + b_ref[...]
        if relu:
            acc = jnp.maximum(acc, 0.0)
        o_ref[...] = acc

    return pl.pallas_call(
        body,
        grid=(n // blk,),
        in_specs=[
            pl.BlockSpec((2, blk, d), lambda i: (0, i, 0)),
            pl.BlockSpec((blk, d), lambda i: (i, 0)),
            pl.BlockSpec((d, d), lambda i: (0, 0)),
            pl.BlockSpec((d, d), lambda i: (0, 0)),
            pl.BlockSpec((1, d), lambda i: (0, 0)),
        ],
        out_specs=pl.BlockSpec((blk, d), lambda i: (i, 0)),
        out_shape=jax.ShapeDtypeStruct((n, d), jnp.float32),
    )(aggp, x, w_rel, w_root, b2d)


def kernel(x, edge_index, W1_rel, b1, W1_root, W2_rel, b2, W2_root):
    n, d = x.shape
    e = edge_index.shape[1]
    assert e % _LANES == 0
    e_rows = e // _LANES
    acc_rows = -(-n // (16 * _LANES)) * (16 * _LANES)

    src = edge_index[0].astype(jnp.int32)
    dst = edge_index[1].astype(jnp.int32)

    seg_sum = _make_sc_segment_sum(n, d, e_rows, acc_rows)
    b1_2d = b1.reshape(1, d)
    b2_2d = b2.reshape(1, d)

    aggp1 = seg_sum(x, src, dst)
    h = _tc_combine(aggp1, x, W1_rel, b1_2d, W1_root, relu=True)
    aggp2 = seg_sum(h, src, dst)
    return _tc_combine(aggp2, h, W2_rel, b2_2d, W2_root, relu=False)
